# Initial kernel scaffold; baseline (speedup 1.0000x reference)
import jax
import jax.numpy as jnp
from jax.experimental import pallas as pl


def kernel(x, weights):
    raise NotImplementedError("write your pallas kernel here")


# speedup vs baseline: 5.7208x; 5.7208x over previous
import jax
import jax.numpy as jnp
from jax.experimental import pallas as pl


def _bmu_kernel(x_ref, wt_ref, out_ref):
    x = x_ref[...]
    wt = wt_ref[...]
    wn = jnp.sum(wt * wt, axis=0, keepdims=True)
    dots = jnp.dot(x, wt,
                   preferred_element_type=jnp.float32,
                   precision=jax.lax.Precision.HIGHEST)
    scores = wn - 2.0 * dots
    m = jnp.min(scores, axis=1, keepdims=True)
    iota = jax.lax.broadcasted_iota(jnp.int32, scores.shape, 1)
    idx = jnp.min(jnp.where(scores == m, iota, scores.shape[1]),
                  axis=1, keepdims=True)
    row = idx // 16
    col = idx - row * 16
    lane = jax.lax.broadcasted_iota(jnp.int32, out_ref.shape, 1)
    out_ref[...] = jnp.where(lane == 0, row, col)


def kernel(x, weights):
    batch, in_size = x.shape
    w_t = weights.reshape(-1, in_size).T
    return pl.pallas_call(
        _bmu_kernel,
        out_shape=jax.ShapeDtypeStruct((batch, 2), jnp.int32),
    )(x, w_t)


# speedup vs baseline: 7.3596x; 1.2865x over previous
import jax
import jax.numpy as jnp
from jax.experimental import pallas as pl


def _bmu_kernel(x_ref, w_ref, out_ref):
    x = x_ref[...]
    wt = w_ref[...].T
    wn = jnp.sum(wt * wt, axis=0, keepdims=True)
    dots = jnp.dot(x, wt,
                   preferred_element_type=jnp.float32,
                   precision=jax.lax.Precision.HIGHEST)
    scores = wn - 2.0 * dots
    m = jnp.min(scores, axis=1, keepdims=True)
    iota = jax.lax.broadcasted_iota(jnp.int32, scores.shape, 1)
    idx = jnp.min(jnp.where(scores == m, iota, scores.shape[1]),
                  axis=1, keepdims=True)
    row = idx // 16
    col = idx - row * 16
    lane = jax.lax.broadcasted_iota(jnp.int32, out_ref.shape, 1)
    out_ref[...] = jnp.where(lane == 0, row, col)


def kernel(x, weights):
    batch, in_size = x.shape
    w_flat = weights.reshape(-1, in_size)
    return pl.pallas_call(
        _bmu_kernel,
        out_shape=jax.ShapeDtypeStruct((batch, 2), jnp.int32),
    )(x, w_flat)
